# baseline (speedup 1.0000x reference)
import functools

import jax
import jax.numpy as jnp
from jax import lax
from jax.experimental import pallas as pl
from jax.experimental.pallas import tpu as pltpu
from jax.experimental.pallas import tpu_sc as plsc

NUM_CLASSES = 1000
HIDDEN = 128
BATCH = 16384

_NC = 2
_NS = 16
_NW = _NC * _NS
_SC_SHARE = 6144
_BPW = _SC_SHARE // _NW
_CHUNK = 96
_NCHUNK = _BPW // _CHUNK


def _sc_body(labels_hbm, table_hbm, out_hbm, idx_v, rows_v, gsem, ssem):
    wid = lax.axis_index("s") * _NC + lax.axis_index("c")
    base = wid * _BPW

    pltpu.sync_copy(labels_hbm.at[pl.ds(wid * _NCHUNK, _NCHUNK), :], idx_v)

    gathers = []
    for j in range(_NCHUNK):
        gathers.append(
            pltpu.async_copy(table_hbm.at[idx_v.at[j]],
                             rows_v.at[pl.ds(j * _CHUNK, _CHUNK), :],
                             gsem))
    scatters = []
    for j in range(_NCHUNK):
        gathers[j].wait()
        scatters.append(
            pltpu.async_copy(rows_v.at[pl.ds(j * _CHUNK, _CHUNK), :],
                             out_hbm.at[pl.ds(base + j * _CHUNK, _CHUNK), :],
                             ssem))
    for s in scatters:
        s.wait()


def _sc_gather(labels_2d, table):
    mesh = plsc.VectorSubcoreMesh(core_axis_name="c", subcore_axis_name="s")
    return pl.kernel(
        _sc_body,
        out_type=jax.ShapeDtypeStruct((_SC_SHARE, HIDDEN), jnp.float32),
        mesh=mesh,
        scratch_types=[
            pltpu.VMEM((_NCHUNK, _CHUNK), jnp.int32),
            pltpu.VMEM((_BPW, HIDDEN), jnp.float32),
            pltpu.SemaphoreType.DMA,
            pltpu.SemaphoreType.DMA,
        ],
    )(labels_2d, table)


_TC_SHARE = BATCH - _SC_SHARE
_BB = 512
_VPAD = 1008


def _tc_body(labels_ref, table_ref, out_ref):
    labels = labels_ref[...]
    classes = lax.broadcasted_iota(jnp.int32, (_BB, _VPAD), 1)
    onehot = (labels == classes).astype(jnp.bfloat16)
    out_ref[...] = jnp.dot(onehot, table_ref[...],
                           preferred_element_type=jnp.float32)


def _tc_embed(labels_2d, table_bf16_padded):
    return pl.pallas_call(
        _tc_body,
        out_shape=jax.ShapeDtypeStruct((_TC_SHARE, HIDDEN), jnp.float32),
        grid=(_TC_SHARE // _BB,),
        in_specs=[
            pl.BlockSpec((_BB, 1), lambda i: (i, 0)),
            pl.BlockSpec((_VPAD, HIDDEN), lambda i: (0, 0)),
        ],
        out_specs=pl.BlockSpec((_BB, HIDDEN), lambda i: (i, 0)),
    )(labels_2d, table_bf16_padded)


@jax.jit
def _embed(labels, table):
    sc_labels = labels[:_SC_SHARE].reshape(_SC_SHARE // _CHUNK, _CHUNK)
    tc_labels = labels[_SC_SHARE:].reshape(_TC_SHARE, 1)
    table_bf16 = jnp.pad(table, ((0, _VPAD - table.shape[0]), (0, 0))
                         ).astype(jnp.bfloat16)
    sc_out = _sc_gather(sc_labels, table)
    tc_out = _tc_embed(tc_labels, table_bf16)
    return jnp.concatenate([sc_out, tc_out], axis=0)


def kernel(labels, embedding_table):
    return _embed(labels, embedding_table)


# speedup vs baseline: 1.5723x; 1.5723x over previous
import functools

import jax
import jax.numpy as jnp
from jax import lax
from jax.experimental import pallas as pl
from jax.experimental.pallas import tpu as pltpu
from jax.experimental.pallas import tpu_sc as plsc

NUM_CLASSES = 1000
HIDDEN = 128
BATCH = 16384

_NC = 2
_NS = 16
_NW = _NC * _NS
_BPW = BATCH // _NW
_CHUNK = 128
_NCHUNK = _BPW // _CHUNK


def _embed_body(labels_hbm, table_hbm, out_hbm, idx_v, rows_v, isem, gsem, ssem):
    wid = lax.axis_index("s") * _NC + lax.axis_index("c")
    base = wid * _BPW

    idx_copies = [
        pltpu.async_copy(labels_hbm.at[pl.ds(wid * _NCHUNK + j, 1), :],
                         idx_v.at[pl.ds(j, 1), :], isem)
        for j in range(_NCHUNK)
    ]

    gathers = []
    for j in range(_NCHUNK):
        idx_copies[j].wait()
        gathers.append(
            pltpu.async_copy(table_hbm.at[idx_v.at[j]],
                             rows_v.at[pl.ds(j * _CHUNK, _CHUNK), :],
                             gsem))
    scatters = []
    for j in range(_NCHUNK):
        gathers[j].wait()
        scatters.append(
            pltpu.async_copy(rows_v.at[pl.ds(j * _CHUNK, _CHUNK), :],
                             out_hbm.at[pl.ds(base + j * _CHUNK, _CHUNK), :],
                             ssem))
    for s in scatters:
        s.wait()


@jax.jit
def _embed(labels, table):
    mesh = plsc.VectorSubcoreMesh(core_axis_name="c", subcore_axis_name="s")
    return pl.kernel(
        _embed_body,
        out_type=jax.ShapeDtypeStruct((BATCH, HIDDEN), jnp.float32),
        mesh=mesh,
        scratch_types=[
            pltpu.VMEM((_NCHUNK, _CHUNK), jnp.int32),
            pltpu.VMEM((_BPW, HIDDEN), jnp.float32),
            pltpu.SemaphoreType.DMA,
            pltpu.SemaphoreType.DMA,
            pltpu.SemaphoreType.DMA,
        ],
    )(labels.reshape(BATCH // _CHUNK, _CHUNK), table)


def kernel(labels, embedding_table):
    return _embed(labels, embedding_table)


# speedup vs baseline: 1.5931x; 1.0133x over previous
import functools

import jax
import jax.numpy as jnp
from jax import lax
from jax.experimental import pallas as pl
from jax.experimental.pallas import tpu as pltpu
from jax.experimental.pallas import tpu_sc as plsc

NUM_CLASSES = 1000
HIDDEN = 128
BATCH = 16384

_NC = 2
_NS = 16
_NW = _NC * _NS
_BPW = BATCH // _NW
_CHUNK = 128
_NCHUNK = _BPW // _CHUNK


def _embed_body(labels_hbm, table_hbm, out_hbm, idx_v, rows_v, gsem, ssem):
    wid = lax.axis_index("s") * _NC + lax.axis_index("c")
    base = wid * _BPW

    pltpu.sync_copy(labels_hbm.at[pl.ds(wid * _NCHUNK, _NCHUNK), :], idx_v)

    gathers = []
    for j in range(_NCHUNK):
        gathers.append(
            pltpu.async_copy(table_hbm.at[idx_v.at[j]],
                             rows_v.at[pl.ds(j * _CHUNK, _CHUNK), :],
                             gsem))
    scatters = []
    for j in range(_NCHUNK):
        gathers[j].wait()
        scatters.append(
            pltpu.async_copy(rows_v.at[pl.ds(j * _CHUNK, _CHUNK), :],
                             out_hbm.at[pl.ds(base + j * _CHUNK, _CHUNK), :],
                             ssem))
    for s in scatters:
        s.wait()


@jax.jit
def _embed(labels, table):
    mesh = plsc.VectorSubcoreMesh(core_axis_name="c", subcore_axis_name="s")
    return pl.kernel(
        _embed_body,
        out_type=jax.ShapeDtypeStruct((BATCH, HIDDEN), jnp.float32),
        mesh=mesh,
        scratch_types=[
            pltpu.VMEM((_NCHUNK, _CHUNK), jnp.int32),
            pltpu.VMEM((_BPW, HIDDEN), jnp.float32),
            pltpu.SemaphoreType.DMA,
            pltpu.SemaphoreType.DMA,
        ],
    )(labels.reshape(BATCH // _CHUNK, _CHUNK), table)


def kernel(labels, embedding_table):
    return _embed(labels, embedding_table)
